# Initial kernel scaffold; baseline (speedup 1.0000x reference)
import jax
import jax.numpy as jnp
from jax.experimental import pallas as pl


def kernel(x, edge_index, edge_weight, W0):
    raise NotImplementedError("write your pallas kernel here")


# speedup vs baseline: 4.0582x; 4.0582x over previous
import functools

import jax
import jax.numpy as jnp
from jax import lax
from jax.experimental import pallas as pl
from jax.experimental.pallas import tpu as pltpu
from jax.experimental.pallas import tpu_sc as plsc

NC = 2
NS = 16
NW = NC * NS
L = 16


def _mm_body(x_ref, w_ref, o_ref):
    o_ref[...] = jnp.dot(x_ref[...], w_ref[...],
                         preferred_element_type=jnp.float32)


def _add_body(a_ref, b_ref, o_ref):
    o_ref[...] = a_ref[...] + b_ref[...]


def _make_sc_scatter(N, D, E, C):
    EP = E // NW
    NCHUNK = EP // C
    RPT = (N // NS) // 8 * 8
    TAIL = N - RPT * NS
    mesh = plsc.VectorSubcoreMesh(core_axis_name="c", subcore_axis_name="s")

    @functools.partial(
        pl.kernel,
        mesh=mesh,
        out_type=jax.ShapeDtypeStruct((NC, N, D), jnp.float32),
        scratch_types=[
            pltpu.VMEM((C,), jnp.int32),
            pltpu.VMEM((C,), jnp.int32),
            pltpu.VMEM((C,), jnp.float32),
            pltpu.VMEM((C, D), jnp.float32),
            pltpu.VMEM_SHARED((N, D), jnp.float32),
            pltpu.SemaphoreType.DMA,
        ],
    )
    def sc_fn(presup_hbm, src_hbm, dst_hbm, w_hbm, zeros_hbm, out_hbm,
              src_v, dst_v, w_v, rows_v, acc, sem):
        cid = lax.axis_index("c")
        sid = lax.axis_index("s")
        wid = sid * NC + cid

        r0 = pl.multiple_of(sid * RPT, 8)
        pltpu.sync_copy(zeros_hbm.at[pl.ds(r0, RPT)], acc.at[pl.ds(r0, RPT)])
        if TAIL:
            @pl.when(sid == 0)
            def _zero_tail():
                t0 = RPT * NS
                pltpu.sync_copy(zeros_hbm.at[pl.ds(t0, TAIL)],
                                acc.at[pl.ds(t0, TAIL)])
        plsc.subcore_barrier()

        base = wid * EP

        def chunk(i, carry):
            off = pl.multiple_of(base + i * C, 8)
            pltpu.sync_copy(src_hbm.at[pl.ds(off, C)], src_v)
            pltpu.sync_copy(dst_hbm.at[pl.ds(off, C)], dst_v)
            pltpu.sync_copy(w_hbm.at[pl.ds(off, C)], w_v)
            pltpu.async_copy(presup_hbm.at[src_v], rows_v, sem).wait()

            def scale(g, c2):
                wg = w_v[pl.ds(pl.multiple_of(g * L, 8), L)]
                for k in range(L):
                    e = g * L + k
                    wb = jnp.full((L,), wg[k])
                    for j in range(D // L):
                        sl = pl.ds(j * L, L)
                        rows_v[e, sl] = rows_v[e, sl] * wb
                return c2

            lax.fori_loop(0, C // L, scale, 0)
            pltpu.sync_copy(rows_v, acc.at[dst_v], add=True)
            return carry

        lax.fori_loop(0, NCHUNK, chunk, 0)
        plsc.subcore_barrier()
        pltpu.sync_copy(acc.at[pl.ds(r0, RPT)],
                        out_hbm.at[cid, pl.ds(r0, RPT)])
        if TAIL:
            @pl.when(sid == 0)
            def _write_tail():
                t0 = RPT * NS
                pltpu.sync_copy(acc.at[pl.ds(t0, TAIL)],
                                out_hbm.at[cid, pl.ds(t0, TAIL)])

    return sc_fn


def kernel(x, edge_index, edge_weight, W0):
    N, D_IN = x.shape
    D_OUT = W0.shape[1]
    E = edge_weight.shape[0]

    BM = 2000
    pre_sup = pl.pallas_call(
        _mm_body,
        grid=(N // BM,),
        in_specs=[
            pl.BlockSpec((BM, D_IN), lambda i: (i, 0)),
            pl.BlockSpec((D_IN, D_OUT), lambda i: (0, 0)),
        ],
        out_specs=pl.BlockSpec((BM, D_OUT), lambda i: (i, 0)),
        out_shape=jax.ShapeDtypeStruct((N, D_OUT), jnp.float32),
    )(x, W0)

    src = edge_index[0]
    dst = edge_index[1]
    zeros = jnp.zeros((N, D_OUT), jnp.float32)

    sc_fn = _make_sc_scatter(N, D_OUT, E, C=80)
    partials = sc_fn(pre_sup, src, dst, edge_weight, zeros)

    out = pl.pallas_call(
        _add_body,
        grid=(N // BM,),
        in_specs=[
            pl.BlockSpec((BM, D_OUT), lambda i: (i, 0)),
            pl.BlockSpec((BM, D_OUT), lambda i: (i, 0)),
        ],
        out_specs=pl.BlockSpec((BM, D_OUT), lambda i: (i, 0)),
        out_shape=jax.ShapeDtypeStruct((N, D_OUT), jnp.float32),
    )(partials[0], partials[1])
    return out
